# baseline (speedup 1.0000x reference)
import functools

import jax
import jax.numpy as jnp
from jax import lax
from jax.experimental import pallas as pl
from jax.experimental.pallas import tpu as pltpu
from jax.experimental.pallas import tpu_sc as plsc

B, C, H, W = 16, 256, 128, 128
N = 4096
TOTAL = B * N
NC, NS, L = 2, 16, 16
NW = NC * NS
PPW = TOTAL // NW
CH = 32
NIDX = 4 * CH
NCHUNK = PPW // CH
NBUF = 2
CV = C // L


def _sc_grid_sample(table, xs, ys):
    mesh = plsc.VectorSubcoreMesh(
        core_axis_name="c", subcore_axis_name="s",
        num_cores=NC, num_subcores=NS)

    @functools.partial(
        pl.kernel,
        out_type=jax.ShapeDtypeStruct((TOTAL, C), jnp.float32),
        mesh=mesh,
        scratch_types=[
            pltpu.VMEM((PPW,), jnp.float32),
            pltpu.VMEM((PPW,), jnp.float32),
            pltpu.VMEM((NBUF, NIDX), jnp.int32),
            pltpu.VMEM((NBUF * NIDX + L,), jnp.float32),
            pltpu.VMEM((NBUF, NIDX, C), jnp.float32),
            pltpu.VMEM((NBUF, CH, C), jnp.float32),
        ] + [pltpu.SemaphoreType.DMA] * (2 * NBUF),
        compiler_params=pltpu.CompilerParams(needs_layout_passes=False),
    )
    def grid_sample_kernel(table_hbm, xs_hbm, ys_hbm, out_hbm,
                           xs_v, ys_v, idx_v, w_v, rows_v, out_v, *sems):
        gsems = list(sems[:NBUF])
        ssems = list(sems[NBUF:])
        wid = lax.axis_index("s") * NC + lax.axis_index("c")
        base_pt = wid * PPW
        img_base = (wid // (N // PPW)) * (H * W)

        pltpu.sync_copy(xs_hbm.at[pl.ds(base_pt, PPW)], xs_v)
        pltpu.sync_copy(ys_hbm.at[pl.ds(base_pt, PPW)], ys_v)

        def compute_idx(g, b):
            for t in range(CH // L):
                off = g * CH + t * L
                xv = xs_v[pl.ds(off, L)]
                yv = ys_v[pl.ds(off, L)]
                ix = (xv + 1.0) * (0.5 * (W - 1))
                iy = (yv + 1.0) * (0.5 * (H - 1))
                ix0 = jnp.minimum(jnp.maximum(ix.astype(jnp.int32), 0), W - 2)
                iy0 = jnp.minimum(jnp.maximum(iy.astype(jnp.int32), 0), H - 2)
                wx1 = ix - ix0.astype(jnp.float32)
                wy1 = iy - iy0.astype(jnp.float32)
                wx0 = 1.0 - wx1
                wy0 = 1.0 - wy1
                r00 = img_base + iy0 * W + ix0
                idx_v[b, pl.ds(0 * CH + t * L, L)] = r00
                idx_v[b, pl.ds(1 * CH + t * L, L)] = r00 + 1
                idx_v[b, pl.ds(2 * CH + t * L, L)] = r00 + W
                idx_v[b, pl.ds(3 * CH + t * L, L)] = r00 + W + 1
                w_v[pl.ds(b * NIDX + 0 * CH + t * L, L)] = wy0 * wx0
                w_v[pl.ds(b * NIDX + 1 * CH + t * L, L)] = wy0 * wx1
                w_v[pl.ds(b * NIDX + 2 * CH + t * L, L)] = wy1 * wx0
                w_v[pl.ds(b * NIDX + 3 * CH + t * L, L)] = wy1 * wx1

        def fire_gather(b):
            pltpu.async_copy(table_hbm.at[idx_v.at[b]], rows_v.at[b], gsems[b])

        def wait_gather(b):
            pltpu.make_async_copy(
                table_hbm.at[idx_v.at[b]], rows_v.at[b], gsems[b]).wait()

        def fire_store(g, b):
            pltpu.async_copy(
                out_v.at[b], out_hbm.at[pl.ds(base_pt + g * CH, CH)], ssems[b])

        def wait_store(b):
            pltpu.make_async_copy(
                out_v.at[b], out_hbm.at[pl.ds(base_pt, CH)], ssems[b]).wait()

        lanes = lax.iota(jnp.int32, L)
        rot_idx = [(lanes + sh) & (L - 1) for sh in (8, 4, 2, 1)]

        def lane_sum(x):
            for idx in rot_idx:
                x = x + x.at[idx].get(mode="promise_in_bounds")
            return x

        def compute_points(b):
            def one_point(p):
                def wbcast(corner):
                    v = w_v[pl.ds(b * NIDX + corner * CH + p, L)]
                    return jnp.full((L,), v[0])

                w00, w01, w10, w11 = (wbcast(0), wbcast(1),
                                      wbcast(2), wbcast(3))
                acc = jnp.zeros((L,), jnp.float32)
                outs = []
                for j in range(CV):
                    sl = pl.ds(j * L, L)
                    o = (rows_v[b, 0 * CH + p, sl] * w00 +
                         rows_v[b, 1 * CH + p, sl] * w01 +
                         rows_v[b, 2 * CH + p, sl] * w10 +
                         rows_v[b, 3 * CH + p, sl] * w11)
                    acc = acc + o * o
                    outs.append(o)
                ssv = jnp.maximum(lane_sum(acc), 1e-24)
                yi = jnp.int32(0x5F3759DF) - (plsc.bitcast(ssv, jnp.int32) >> 1)
                y = plsc.bitcast(yi, jnp.float32)
                half = ssv * 0.5
                for _ in range(2):
                    y = y * (1.5 - half * y * y)
                for j in range(CV):
                    out_v[b, p, pl.ds(j * L, L)] = outs[j] * y

            def body(p2, _):
                one_point(2 * p2)
                one_point(2 * p2 + 1)
                return 0

            lax.fori_loop(0, CH // 2, body, 0)

        for b in range(NBUF):
            compute_idx(b, b)
            fire_gather(b)

        def chunk_iter(m, _):
            for b in range(NBUF):
                g = m * NBUF + b
                wait_gather(b)

                @pl.when(g >= NBUF)
                def _():
                    wait_store(b)

                compute_points(b)

                @pl.when(g + NBUF < NCHUNK)
                def _():
                    compute_idx(g + NBUF, b)
                    fire_gather(b)

                fire_store(g, b)
            return 0

        lax.fori_loop(0, NCHUNK // NBUF, chunk_iter, 0)
        for b in range(NBUF):
            wait_store(b)

    return grid_sample_kernel(table, xs, ys)


def _tc_transpose(desc):
    nb = desc.shape[0]

    def body(in_ref, out_ref):
        out_ref[...] = in_ref[0].T

    return pl.pallas_call(
        body,
        grid=(nb, H),
        in_specs=[pl.BlockSpec((1, C, W), lambda b, y: (b, 0, y))],
        out_specs=pl.BlockSpec((W, C), lambda b, y: (b * H + y, 0)),
        out_shape=jax.ShapeDtypeStruct((nb * H * W, C), jnp.float32),
    )(desc.reshape(nb, C, H * W))


def kernel(descriptors, keypoints_norm):
    table = _tc_transpose(descriptors)
    kp = keypoints_norm.reshape(TOTAL, 2)
    xs = kp[:, 0]
    ys = kp[:, 1]
    out_t = _sc_grid_sample(table, xs, ys)
    return out_t.reshape(B, N, C).transpose(0, 2, 1)


# speedup vs baseline: 4.3301x; 4.3301x over previous
import functools

import jax
import jax.numpy as jnp
from jax import lax
from jax.experimental import pallas as pl
from jax.experimental.pallas import tpu as pltpu
from jax.experimental.pallas import tpu_sc as plsc

B, C, H, W = 16, 256, 128, 128
N = 4096
TOTAL = B * N
NC, NS, L = 2, 16, 16
NW = NC * NS
PPW = TOTAL // NW
CH = 32
NIDX = 4 * CH
NCHUNK = PPW // CH
NBUF = 2
CV = C // L


def _sc_grid_sample(table, xs, ys):
    mesh = plsc.VectorSubcoreMesh(
        core_axis_name="c", subcore_axis_name="s",
        num_cores=NC, num_subcores=NS)

    @functools.partial(
        pl.kernel,
        out_type=jax.ShapeDtypeStruct((TOTAL, C), jnp.float32),
        mesh=mesh,
        scratch_types=[
            pltpu.VMEM((PPW,), jnp.float32),
            pltpu.VMEM((PPW,), jnp.float32),
            pltpu.VMEM((NBUF, NIDX), jnp.int32),
            pltpu.VMEM((NBUF * NIDX + L,), jnp.float32),
            pltpu.VMEM((NBUF, NIDX, C), jnp.float32),
            pltpu.VMEM((NBUF, CH, C), jnp.float32),
        ] + [pltpu.SemaphoreType.DMA] * (2 * NBUF),
        compiler_params=pltpu.CompilerParams(needs_layout_passes=False),
    )
    def grid_sample_kernel(table_hbm, xs_hbm, ys_hbm, out_hbm,
                           xs_v, ys_v, idx_v, w_v, rows_v, out_v, *sems):
        gsems = list(sems[:NBUF])
        ssems = list(sems[NBUF:])
        wid = lax.axis_index("s") * NC + lax.axis_index("c")
        base_pt = wid * PPW
        img_base = (wid // (N // PPW)) * (H * W)

        pltpu.sync_copy(xs_hbm.at[pl.ds(base_pt, PPW)], xs_v)
        pltpu.sync_copy(ys_hbm.at[pl.ds(base_pt, PPW)], ys_v)

        def compute_idx(g, b):
            for t in range(CH // L):
                off = g * CH + t * L
                xv = xs_v[pl.ds(off, L)]
                yv = ys_v[pl.ds(off, L)]
                ix = (xv + 1.0) * (0.5 * (W - 1))
                iy = (yv + 1.0) * (0.5 * (H - 1))
                ix0 = jnp.minimum(jnp.maximum(ix.astype(jnp.int32), 0), W - 2)
                iy0 = jnp.minimum(jnp.maximum(iy.astype(jnp.int32), 0), H - 2)
                wx1 = ix - ix0.astype(jnp.float32)
                wy1 = iy - iy0.astype(jnp.float32)
                wx0 = 1.0 - wx1
                wy0 = 1.0 - wy1
                r00 = img_base + iy0 * W + ix0
                idx_v[b, pl.ds(0 * CH + t * L, L)] = r00
                idx_v[b, pl.ds(1 * CH + t * L, L)] = r00 + 1
                idx_v[b, pl.ds(2 * CH + t * L, L)] = r00 + W
                idx_v[b, pl.ds(3 * CH + t * L, L)] = r00 + W + 1
                w_v[pl.ds(b * NIDX + 0 * CH + t * L, L)] = wy0 * wx0
                w_v[pl.ds(b * NIDX + 1 * CH + t * L, L)] = wy0 * wx1
                w_v[pl.ds(b * NIDX + 2 * CH + t * L, L)] = wy1 * wx0
                w_v[pl.ds(b * NIDX + 3 * CH + t * L, L)] = wy1 * wx1

        def fire_gather(b):
            pltpu.async_copy(table_hbm.at[idx_v.at[b]], rows_v.at[b], gsems[b])

        def wait_gather(b):
            pltpu.make_async_copy(
                table_hbm.at[idx_v.at[b]], rows_v.at[b], gsems[b]).wait()

        def fire_store(g, b):
            pltpu.async_copy(
                out_v.at[b], out_hbm.at[pl.ds(base_pt + g * CH, CH)], ssems[b])

        def wait_store(b):
            pltpu.make_async_copy(
                out_v.at[b], out_hbm.at[pl.ds(base_pt, CH)], ssems[b]).wait()

        lanes = lax.iota(jnp.int32, L)
        rot_idx = [(lanes + sh) & (L - 1) for sh in (8, 4, 2, 1)]

        def lane_sum(x):
            for idx in rot_idx:
                x = x + x.at[idx].get(mode="promise_in_bounds")
            return x

        def compute_points(b):
            def one_point(p):
                def wbcast(corner):
                    v = w_v[pl.ds(b * NIDX + corner * CH + p, L)]
                    return jnp.full((L,), v[0])

                w00, w01, w10, w11 = (wbcast(0), wbcast(1),
                                      wbcast(2), wbcast(3))
                acc = jnp.zeros((L,), jnp.float32)
                outs = []
                for j in range(CV):
                    sl = pl.ds(j * L, L)
                    o = (rows_v[b, 0 * CH + p, sl] * w00 +
                         rows_v[b, 1 * CH + p, sl] * w01 +
                         rows_v[b, 2 * CH + p, sl] * w10 +
                         rows_v[b, 3 * CH + p, sl] * w11)
                    acc = acc + o * o
                    outs.append(o)
                ssv = jnp.maximum(lane_sum(acc), 1e-24)
                yi = jnp.int32(0x5F3759DF) - (plsc.bitcast(ssv, jnp.int32) >> 1)
                y = plsc.bitcast(yi, jnp.float32)
                half = ssv * 0.5
                for _ in range(2):
                    y = y * (1.5 - half * y * y)
                for j in range(CV):
                    out_v[b, p, pl.ds(j * L, L)] = outs[j] * y

            @plsc.parallel_loop(0, CH, 1, unroll=2)
            def _(p):
                one_point(p)

        for b in range(NBUF):
            compute_idx(b, b)
            fire_gather(b)

        def chunk_iter(m, _):
            for b in range(NBUF):
                g = m * NBUF + b
                wait_gather(b)

                @pl.when(g >= NBUF)
                def _():
                    wait_store(b)

                compute_points(b)

                @pl.when(g + NBUF < NCHUNK)
                def _():
                    compute_idx(g + NBUF, b)
                    fire_gather(b)

                fire_store(g, b)
            return 0

        lax.fori_loop(0, NCHUNK // NBUF, chunk_iter, 0)
        for b in range(NBUF):
            wait_store(b)

    return grid_sample_kernel(table, xs, ys)


def _tc_transpose(desc):
    nb = desc.shape[0]

    def body(in_ref, out_ref):
        out_ref[...] = in_ref[0].T

    return pl.pallas_call(
        body,
        grid=(nb, H),
        in_specs=[pl.BlockSpec((1, C, W), lambda b, y: (b, 0, y))],
        out_specs=pl.BlockSpec((W, C), lambda b, y: (b * H + y, 0)),
        out_shape=jax.ShapeDtypeStruct((nb * H * W, C), jnp.float32),
    )(desc.reshape(nb, C, H * W))


def kernel(descriptors, keypoints_norm):
    table = descriptors.transpose(0, 2, 3, 1).reshape(B * H * W, C)
    kp = keypoints_norm.reshape(TOTAL, 2)
    xs = kp[:, 0]
    ys = kp[:, 1]
    out_t = _sc_grid_sample(table, xs, ys)
    return out_t.reshape(B, N, C).transpose(0, 2, 1)


# speedup vs baseline: 4.3311x; 1.0002x over previous
import functools

import jax
import jax.numpy as jnp
from jax import lax
from jax.experimental import pallas as pl
from jax.experimental.pallas import tpu as pltpu
from jax.experimental.pallas import tpu_sc as plsc

B, C, H, W = 16, 256, 128, 128
N = 4096
TOTAL = B * N
NC, NS, L = 2, 16, 16
NW = NC * NS
PPW = TOTAL // NW
CH = 32
NIDX = 4 * CH
NCHUNK = PPW // CH
NBUF = 2
CV = C // L


def _sc_grid_sample(table, xs, ys):
    mesh = plsc.VectorSubcoreMesh(
        core_axis_name="c", subcore_axis_name="s",
        num_cores=NC, num_subcores=NS)

    @functools.partial(
        pl.kernel,
        out_type=jax.ShapeDtypeStruct((TOTAL, C), jnp.float32),
        mesh=mesh,
        scratch_types=[
            pltpu.VMEM((PPW,), jnp.float32),
            pltpu.VMEM((PPW,), jnp.float32),
            pltpu.VMEM((NIDX,), jnp.int32),
            pltpu.VMEM((NIDX,), jnp.int32),
            pltpu.VMEM((NBUF * NIDX + L,), jnp.float32),
            pltpu.VMEM((NBUF, NIDX, C), jnp.float32),
            pltpu.VMEM((NBUF, CH, C), jnp.float32),
        ] + [pltpu.SemaphoreType.DMA] * (2 * NBUF),
        compiler_params=pltpu.CompilerParams(needs_layout_passes=False),
    )
    def grid_sample_kernel(table_hbm, xs_hbm, ys_hbm, out_hbm,
                           xs_v, ys_v, idx_v0, idx_v1, w_v, rows_v, out_v,
                           *sems):
        idxs = [idx_v0, idx_v1]
        gsems = list(sems[:NBUF])
        ssems = list(sems[NBUF:])
        wid = lax.axis_index("s") * NC + lax.axis_index("c")
        base_pt = wid * PPW
        img_base = (wid // (N // PPW)) * (H * W)

        pltpu.sync_copy(xs_hbm.at[pl.ds(base_pt, PPW)], xs_v)
        pltpu.sync_copy(ys_hbm.at[pl.ds(base_pt, PPW)], ys_v)

        def compute_idx(g, b):
            for t in range(CH // L):
                off = g * CH + t * L
                xv = xs_v[pl.ds(off, L)]
                yv = ys_v[pl.ds(off, L)]
                ix = (xv + 1.0) * (0.5 * (W - 1))
                iy = (yv + 1.0) * (0.5 * (H - 1))
                ix0 = jnp.minimum(jnp.maximum(ix.astype(jnp.int32), 0), W - 2)
                iy0 = jnp.minimum(jnp.maximum(iy.astype(jnp.int32), 0), H - 2)
                wx1 = ix - ix0.astype(jnp.float32)
                wy1 = iy - iy0.astype(jnp.float32)
                wx0 = 1.0 - wx1
                wy0 = 1.0 - wy1
                r00 = img_base + iy0 * W + ix0
                idxs[b][pl.ds(0 * CH + t * L, L)] = r00
                idxs[b][pl.ds(1 * CH + t * L, L)] = r00 + 1
                idxs[b][pl.ds(2 * CH + t * L, L)] = r00 + W
                idxs[b][pl.ds(3 * CH + t * L, L)] = r00 + W + 1
                w_v[pl.ds(b * NIDX + 0 * CH + t * L, L)] = wy0 * wx0
                w_v[pl.ds(b * NIDX + 1 * CH + t * L, L)] = wy0 * wx1
                w_v[pl.ds(b * NIDX + 2 * CH + t * L, L)] = wy1 * wx0
                w_v[pl.ds(b * NIDX + 3 * CH + t * L, L)] = wy1 * wx1

        def fire_gather(b):
            pltpu.async_copy(table_hbm.at[idxs[b]], rows_v.at[b], gsems[b])

        def wait_gather(b):
            pltpu.make_async_copy(
                table_hbm.at[idxs[b]], rows_v.at[b], gsems[b]).wait()

        def fire_store(g, b):
            pltpu.async_copy(
                out_v.at[b], out_hbm.at[pl.ds(base_pt + g * CH, CH)], ssems[b])

        def wait_store(b):
            pltpu.make_async_copy(
                out_v.at[b], out_hbm.at[pl.ds(base_pt, CH)], ssems[b]).wait()

        lanes = lax.iota(jnp.int32, L)
        rot_idx = [(lanes + sh) & (L - 1) for sh in (8, 4, 2, 1)]

        def lane_sum(x):
            for idx in rot_idx:
                x = x + x.at[idx].get(mode="promise_in_bounds")
            return x

        def compute_points(b):
            def one_point(p):
                def wbcast(corner):
                    v = w_v[pl.ds(b * NIDX + corner * CH + p, L)]
                    return jnp.full((L,), v[0])

                w00, w01, w10, w11 = (wbcast(0), wbcast(1),
                                      wbcast(2), wbcast(3))
                acc = jnp.zeros((L,), jnp.float32)
                outs = []
                for j in range(CV):
                    sl = pl.ds(j * L, L)
                    o = (rows_v[b, 0 * CH + p, sl] * w00 +
                         rows_v[b, 1 * CH + p, sl] * w01 +
                         rows_v[b, 2 * CH + p, sl] * w10 +
                         rows_v[b, 3 * CH + p, sl] * w11)
                    acc = acc + o * o
                    outs.append(o)
                ssv = jnp.maximum(lane_sum(acc), 1e-24)
                yi = jnp.int32(0x5F3759DF) - (plsc.bitcast(ssv, jnp.int32) >> 1)
                y = plsc.bitcast(yi, jnp.float32)
                half = ssv * 0.5
                for _ in range(2):
                    y = y * (1.5 - half * y * y)
                for j in range(CV):
                    out_v[b, p, pl.ds(j * L, L)] = outs[j] * y

            @plsc.parallel_loop(0, CH, 1, unroll=2)
            def _(p):
                one_point(p)

        for b in range(NBUF):
            compute_idx(b, b)
            fire_gather(b)

        def chunk_iter(m, _):
            for b in range(NBUF):
                g = m * NBUF + b
                wait_gather(b)

                @pl.when(g >= NBUF)
                def _():
                    wait_store(b)

                compute_points(b)

                @pl.when(g + NBUF < NCHUNK)
                def _():
                    compute_idx(g + NBUF, b)
                    fire_gather(b)

                fire_store(g, b)
            return 0

        lax.fori_loop(0, NCHUNK // NBUF, chunk_iter, 0)
        for b in range(NBUF):
            wait_store(b)

    return grid_sample_kernel(table, xs, ys)


def kernel(descriptors, keypoints_norm):
    table = descriptors.transpose(0, 2, 3, 1).reshape(B * H * W, C)
    kp = keypoints_norm.reshape(TOTAL, 2)
    xs = kp[:, 0]
    ys = kp[:, 1]
    out_t = _sc_grid_sample(table, xs, ys)
    return out_t.reshape(B, N, C).transpose(0, 2, 1)
